# baseline (speedup 1.0000x reference)
import functools

import jax
import jax.numpy as jnp
from jax import lax
from jax.experimental import pallas as pl
from jax.experimental.pallas import tpu as pltpu
from jax.experimental.pallas import tpu_sc as plsc

NN = 10000
EE = 320000
DD = 128
EDIMK = 16

ROW_BLK = 1000
EA_BLK = 8000

NTILES = 32
NPT = 320
NPAD = NTILES * NPT
CHUNK = 4000
NVEC = CHUNK // 16
NCHUNK = EE // CHUNK
BATCH = 128
MCAP = CHUNK + BATCH


def _sc_segmax_body(dst_hbm, src_hbm, b_hbm, ea_hbm, s_hbm,
                    acc, dstbuf, srcbuf, mdl, msrc, meid,
                    rows_b, rows_e, sem0, sem1):
    cid = lax.axis_index("c")
    sid = lax.axis_index("s")
    wid = sid * 2 + cid
    lo = wid * NPT

    minus_inf = jnp.full((16,), -jnp.inf, dtype=jnp.float32)

    @pl.loop(0, NPT)
    def _(r):
        @pl.loop(0, DD, step=16)
        def _(c):
            acc[r, pl.ds(c, 16)] = minus_inf

    zeros16 = jnp.zeros((16,), dtype=jnp.int32)

    @pl.loop(0, MCAP, step=16)
    def _(i):
        msrc[pl.ds(i, 16)] = zeros16
        meid[pl.ds(i, 16)] = zeros16

    lane = lax.iota(jnp.int32, 16)

    @pl.loop(0, NCHUNK)
    def _(ci):
        e0 = ci * CHUNK
        pltpu.sync_copy(dst_hbm.at[pl.ds(e0, CHUNK)], dstbuf)
        pltpu.sync_copy(src_hbm.at[pl.ds(e0, CHUNK)], srcbuf)

        def scan_body(v, count):
            sl = pl.ds(v * 16, 16)
            dl = dstbuf[sl] - lo
            mask = (dl >= 0) & (dl < NPT)
            csl = pl.ds(count, 16)
            plsc.store_compressed(mdl.at[csl], dl, mask=mask)
            plsc.store_compressed(msrc.at[csl], srcbuf[sl], mask=mask)
            plsc.store_compressed(meid.at[csl], lane + (e0 + v * 16), mask=mask)
            return count + jnp.sum(mask.astype(jnp.int32), axis=0)

        count = lax.fori_loop(0, NVEC, scan_body, jnp.int32(0), unroll=False)

        def batch_body(b, _):
            base = b * BATCH
            bs = jnp.minimum(count - base, BATCH)
            pltpu.async_copy(b_hbm.at[msrc.at[pl.ds(base, BATCH)]], rows_b,
                             sem0)
            pltpu.async_copy(ea_hbm.at[meid.at[pl.ds(base, BATCH)]], rows_e,
                             sem1)
            pltpu.make_async_copy(b_hbm.at[msrc.at[pl.ds(base, BATCH)]],
                                  rows_b, sem0).wait()
            pltpu.make_async_copy(ea_hbm.at[meid.at[pl.ds(base, BATCH)]],
                                  rows_e, sem1).wait()

            def edge_body(j, carry):
                d = mdl[pl.ds(base + j, 16)][0]
                for k in range(DD // 16):
                    sl = pl.ds(k * 16, 16)
                    row = rows_b[j, sl] + rows_e[j, sl]
                    acc[d, sl] = jnp.maximum(acc[d, sl], row)
                return carry

            lax.fori_loop(0, bs, edge_body, jnp.int32(0))
            return _

        nb = (count + (BATCH - 1)) // BATCH
        lax.fori_loop(0, nb, batch_body, jnp.int32(0))

    pltpu.sync_copy(acc, s_hbm.at[pl.ds(lo, NPT)])


def _sc_segmax(dst, src, b_tab, ea_proj):
    import dataclasses
    mesh = plsc.VectorSubcoreMesh(core_axis_name="c", subcore_axis_name="s")
    f32 = jnp.float32
    cp = pltpu.CompilerParams()
    if "needs_layout_passes" in pltpu.CompilerParams.__dataclass_fields__:
        cp = dataclasses.replace(cp, needs_layout_passes=False)
    run = pl.kernel(
        _sc_segmax_body,
        compiler_params=cp,
        out_type=jax.ShapeDtypeStruct((NPAD, DD), f32),
        mesh=mesh,
        scratch_types=[
            pltpu.VMEM((NPT, DD), f32),
            pltpu.VMEM((CHUNK,), jnp.int32),
            pltpu.VMEM((CHUNK,), jnp.int32),
            pltpu.VMEM((MCAP,), jnp.int32),
            pltpu.VMEM((MCAP,), jnp.int32),
            pltpu.VMEM((MCAP,), jnp.int32),
            pltpu.VMEM((BATCH, DD), f32),
            pltpu.VMEM((BATCH, DD), f32),
            pltpu.SemaphoreType.DMA,
            pltpu.SemaphoreType.DMA,
        ],
    )
    return run(dst, src, b_tab, ea_proj)


def _enc_ab_body(x_ref, ph_ref, wxa_ref, wxb_ref, benc_ref, wma_ref, wmb_ref,
                 z_ref, a_ref, b_ref):
    z = jnp.maximum(
        jnp.dot(x_ref[...], wxa_ref[...], preferred_element_type=jnp.float32)
        + jnp.dot(ph_ref[...], wxb_ref[...], preferred_element_type=jnp.float32)
        + benc_ref[...], 0.0)
    z_ref[...] = z
    a_ref[...] = jnp.dot(z, wma_ref[...], preferred_element_type=jnp.float32)
    b_ref[...] = jnp.dot(z, wmb_ref[...], preferred_element_type=jnp.float32)


def _ea_body(ea_ref, wme_ref, out_ref):
    out_ref[...] = jnp.dot(ea_ref[...], wme_ref[...],
                           preferred_element_type=jnp.float32)


def _epilogue_body(z_ref, a_ref, s_ref, bm_ref, wua_ref, wub_ref, bu_ref,
                   wdz_ref, wdh_ref, bdec_ref, wtau_ref, btau_ref,
                   h_ref, y_ref, tau_ref, hsum_ref):
    i = pl.program_id(0)
    aggr = jnp.maximum(a_ref[...] + bm_ref[...] + s_ref[...], 0.0)
    h = jnp.maximum(
        jnp.dot(z_ref[...], wua_ref[...], preferred_element_type=jnp.float32)
        + jnp.dot(aggr, wub_ref[...], preferred_element_type=jnp.float32)
        + bu_ref[...], 0.0)
    h_ref[...] = h
    y_ref[...] = (
        jnp.dot(z_ref[...], wdz_ref[...], preferred_element_type=jnp.float32)
        + jnp.dot(h, wdh_ref[...], preferred_element_type=jnp.float32)
        + bdec_ref[...])

    @pl.when(i == 0)
    def _():
        hsum_ref[...] = jnp.zeros_like(hsum_ref)

    hsum_ref[...] += jnp.sum(h, axis=0, keepdims=True)

    @pl.when(i == pl.num_programs(0) - 1)
    def _():
        hmean = hsum_ref[...] * (1.0 / NN)
        tau_ref[...] = (
            jnp.dot(hmean, wtau_ref[...], preferred_element_type=jnp.float32)
            + btau_ref[...])


def kernel(x, pre_h, edge_index, edge_attr, W_enc, b_enc, W_M, b_M, W_U, b_U,
           W_dec, b_dec, W_tau, b_tau):
    src = edge_index[0]
    dst = edge_index[1]

    wxa = W_enc[:, :DD].T
    wxb = W_enc[:, DD:].T
    wma = W_M[:, :DD].T
    wmb = W_M[:, DD:2 * DD].T
    wme = W_M[:, 2 * DD:].T
    wua = W_U[:, :DD].T
    wub = W_U[:, DD:].T
    wdz = W_dec[:, :DD].T
    wdh = W_dec[:, DD:].T
    wtau = W_tau.T

    n_blocks = NN // ROW_BLK
    row_spec = pl.BlockSpec((ROW_BLK, DD), lambda i: (i, 0))
    full_spec = pl.BlockSpec((DD, DD), lambda i: (0, 0))
    bias_spec = pl.BlockSpec((1, DD), lambda i: (0, 0))

    z, a_tab, b_tab = pl.pallas_call(
        _enc_ab_body,
        grid=(n_blocks,),
        in_specs=[row_spec, row_spec, full_spec, full_spec, bias_spec,
                  full_spec, full_spec],
        out_specs=[row_spec, row_spec, row_spec],
        out_shape=[jax.ShapeDtypeStruct((NN, DD), jnp.float32)] * 3,
    )(x, pre_h, wxa, wxb, b_enc.reshape(1, DD), wma, wmb)

    ea_proj = pl.pallas_call(
        _ea_body,
        grid=(EE // EA_BLK,),
        in_specs=[pl.BlockSpec((EA_BLK, EDIMK), lambda i: (i, 0)),
                  pl.BlockSpec((EDIMK, DD), lambda i: (0, 0))],
        out_specs=pl.BlockSpec((EA_BLK, DD), lambda i: (i, 0)),
        out_shape=jax.ShapeDtypeStruct((EE, DD), jnp.float32),
    )(edge_attr, wme)

    s_pad = _sc_segmax(dst, src, b_tab, ea_proj)
    s_tab = s_pad[:NN]

    col_spec = pl.BlockSpec((DD, 1), lambda i: (0, 0))
    h, y, tau = pl.pallas_call(
        _epilogue_body,
        grid=(n_blocks,),
        in_specs=[row_spec, row_spec, row_spec, bias_spec,
                  full_spec, full_spec, bias_spec,
                  col_spec, col_spec, pl.BlockSpec((1, 1), lambda i: (0, 0)),
                  col_spec, pl.BlockSpec((1, 1), lambda i: (0, 0))],
        out_specs=[row_spec, pl.BlockSpec((ROW_BLK, 1), lambda i: (i, 0)),
                   pl.BlockSpec((1, 1), lambda i: (0, 0))],
        out_shape=[jax.ShapeDtypeStruct((NN, DD), jnp.float32),
                   jax.ShapeDtypeStruct((NN, 1), jnp.float32),
                   jax.ShapeDtypeStruct((1, 1), jnp.float32)],
        scratch_shapes=[pltpu_vmem((1, DD), jnp.float32)],
    )(z, a_tab, s_tab, b_M.reshape(1, DD), wua, wub, b_U.reshape(1, DD),
      wdz, wdh, b_dec.reshape(1, 1), wtau, b_tau.reshape(1, 1))

    return (h, y, tau)


def pltpu_vmem(shape, dtype):
    from jax.experimental.pallas import tpu as pltpu
    return pltpu.VMEM(shape, dtype)


# speedup vs baseline: 3.4013x; 3.4013x over previous
import functools

import jax
import jax.numpy as jnp
from jax import lax
from jax.experimental import pallas as pl
from jax.experimental.pallas import tpu as pltpu
from jax.experimental.pallas import tpu_sc as plsc

NN = 10000
EE = 320000
DD = 128
EDIMK = 16

ROW_BLK = 1000
EA_BLK = 8000

NPT = 640
NPAD = 16 * NPT
EHALF = EE // 2
CHUNK = 1600
NVEC = CHUNK // 16
NCHUNK = EHALF // CHUNK
BATCH = 64
MCAP = CHUNK + BATCH + 16
TRASH = NPT


def _sc_segmax_body(dst_hbm, src_hbm, b_hbm, ea_hbm, s_hbm,
                    acc, dbuf0, sbuf0, dbuf1, sbuf1, mdl, msrc, meid,
                    rows_b0, rows_e0, rows_b1, rows_e1,
                    sd0, ss0, sd1, ss1, sb0, se0, sb1, se1):
    cid = lax.axis_index("c")
    sid = lax.axis_index("s")
    lo = sid * NPT
    ebase = cid * EHALF

    minus_inf = jnp.full((16,), -jnp.inf, dtype=jnp.float32)

    @pl.loop(0, NPT + 1)
    def _(r):
        @pl.loop(0, DD, step=16)
        def _(c):
            acc[r, pl.ds(c, 16)] = minus_inf

    zeros16 = jnp.zeros((16,), dtype=jnp.int32)

    @pl.loop(0, MCAP, step=16)
    def _(i):
        msrc[pl.ds(i, 16)] = zeros16
        meid[pl.ds(i, 16)] = zeros16

    lane = lax.iota(jnp.int32, 16)
    trash16 = jnp.full((16,), TRASH, dtype=jnp.int32)

    def start_stream(ci, dbuf, sbuf, semd, sems):
        e0 = ebase + ci * CHUNK
        pltpu.async_copy(dst_hbm.at[pl.ds(e0, CHUNK)], dbuf, semd)
        pltpu.async_copy(src_hbm.at[pl.ds(e0, CHUNK)], sbuf, sems)

    def wait_stream(ci, dbuf, sbuf, semd, sems):
        e0 = ebase + ci * CHUNK
        pltpu.make_async_copy(dst_hbm.at[pl.ds(e0, CHUNK)], dbuf, semd).wait()
        pltpu.make_async_copy(src_hbm.at[pl.ds(e0, CHUNK)], sbuf, sems).wait()

    def start_gather(base, rb, re, semb, seme):
        pltpu.async_copy(b_hbm.at[msrc.at[pl.ds(base, BATCH)]], rb, semb)
        pltpu.async_copy(ea_hbm.at[meid.at[pl.ds(base, BATCH)]], re, seme)

    def wait_gather(base, rb, re, semb, seme):
        pltpu.make_async_copy(b_hbm.at[msrc.at[pl.ds(base, BATCH)]], rb,
                              semb).wait()
        pltpu.make_async_copy(ea_hbm.at[meid.at[pl.ds(base, BATCH)]], re,
                              seme).wait()

    def rmw_batch(base, rb, re):
        @pl.loop(0, BATCH // 16)
        def _(g):
            dvec = mdl[pl.ds(base + g * 16, 16)]
            for l in range(16):
                d = dvec[l]
                r = g * 16 + l
                for k in range(DD // 16):
                    sl = pl.ds(k * 16, 16)
                    row = rb[r, sl] + re[r, sl]
                    acc[d, sl] = jnp.maximum(acc[d, sl], row)

    def process_chunk(ci, dbuf, sbuf):
        e0 = ebase + ci * CHUNK

        @plsc.parallel_loop(0, NVEC, carry=jnp.int32(0))
        def count(v, cnt):
            sl = pl.ds(v * 16, 16)
            dl = dbuf[sl] - lo
            mask = (dl >= 0) & (dl < NPT)
            csl = pl.ds(cnt, 16)
            plsc.store_compressed(mdl.at[csl], dl, mask=mask)
            plsc.store_compressed(msrc.at[csl], sbuf[sl], mask=mask)
            plsc.store_compressed(meid.at[csl], lane + (e0 + v * 16),
                                  mask=mask)
            return cnt + plsc.all_reduce_population_count(mask)[0]

        for i in range(BATCH // 16):
            mdl[pl.ds(count + i * 16, 16)] = trash16

        nb = (count + (BATCH - 1)) // BATCH

        @pl.when(nb > 0)
        def _():
            start_gather(0, rows_b0, rows_e0, sb0, se0)

        def bb_body(bb, carry):
            b0 = 2 * bb
            b1 = b0 + 1
            base0 = b0 * BATCH
            base1 = b1 * BATCH
            wait_gather(base0, rows_b0, rows_e0, sb0, se0)

            @pl.when(b1 < nb)
            def _():
                start_gather(base1, rows_b1, rows_e1, sb1, se1)

            rmw_batch(base0, rows_b0, rows_e0)

            @pl.when(b1 < nb)
            def _():
                wait_gather(base1, rows_b1, rows_e1, sb1, se1)

                @pl.when(b1 + 1 < nb)
                def _():
                    start_gather(base1 + BATCH, rows_b0, rows_e0, sb0, se0)

                rmw_batch(base1, rows_b1, rows_e1)

            return carry

        lax.fori_loop(0, (nb + 1) // 2, bb_body, jnp.int32(0))

    start_stream(0, dbuf0, sbuf0, sd0, ss0)

    @pl.loop(0, NCHUNK // 2)
    def _(ci2):
        ca = 2 * ci2
        cb = ca + 1
        wait_stream(ca, dbuf0, sbuf0, sd0, ss0)
        start_stream(cb, dbuf1, sbuf1, sd1, ss1)
        process_chunk(ca, dbuf0, sbuf0)
        wait_stream(cb, dbuf1, sbuf1, sd1, ss1)

        @pl.when(ca + 2 < NCHUNK)
        def _():
            start_stream(ca + 2, dbuf0, sbuf0, sd0, ss0)

        process_chunk(cb, dbuf1, sbuf1)

    pltpu.sync_copy(acc.at[pl.ds(0, NPT)],
                    s_hbm.at[cid].at[pl.ds(lo, NPT)])


def _sc_segmax(dst, src, b_tab, ea_proj):
    import dataclasses
    mesh = plsc.VectorSubcoreMesh(core_axis_name="c", subcore_axis_name="s")
    f32 = jnp.float32
    cp = pltpu.CompilerParams()
    if "needs_layout_passes" in pltpu.CompilerParams.__dataclass_fields__:
        cp = dataclasses.replace(cp, needs_layout_passes=False)
    i32 = jnp.int32
    run = pl.kernel(
        _sc_segmax_body,
        compiler_params=cp,
        out_type=jax.ShapeDtypeStruct((2, NPAD, DD), f32),
        mesh=mesh,
        scratch_types=[
            pltpu.VMEM((NPT + 1, DD), f32),
            pltpu.VMEM((CHUNK,), i32),
            pltpu.VMEM((CHUNK,), i32),
            pltpu.VMEM((CHUNK,), i32),
            pltpu.VMEM((CHUNK,), i32),
            pltpu.VMEM((MCAP,), i32),
            pltpu.VMEM((MCAP,), i32),
            pltpu.VMEM((MCAP,), i32),
            pltpu.VMEM((BATCH, DD), f32),
            pltpu.VMEM((BATCH, DD), f32),
            pltpu.VMEM((BATCH, DD), f32),
            pltpu.VMEM((BATCH, DD), f32),
        ] + [pltpu.SemaphoreType.DMA] * 8,
    )
    return run(dst, src, b_tab, ea_proj)


def _enc_ab_body(x_ref, ph_ref, wxa_ref, wxb_ref, benc_ref, wma_ref, wmb_ref,
                 z_ref, a_ref, b_ref):
    z = jnp.maximum(
        jnp.dot(x_ref[...], wxa_ref[...], preferred_element_type=jnp.float32)
        + jnp.dot(ph_ref[...], wxb_ref[...], preferred_element_type=jnp.float32)
        + benc_ref[...], 0.0)
    z_ref[...] = z
    a_ref[...] = jnp.dot(z, wma_ref[...], preferred_element_type=jnp.float32)
    b_ref[...] = jnp.dot(z, wmb_ref[...], preferred_element_type=jnp.float32)


def _ea_body(ea_ref, wme_ref, out_ref):
    out_ref[...] = jnp.dot(ea_ref[...], wme_ref[...],
                           preferred_element_type=jnp.float32)


def _epilogue_body(z_ref, a_ref, s0_ref, s1_ref, bm_ref, wua_ref, wub_ref,
                   bu_ref, wdz_ref, wdh_ref, bdec_ref, wtau_ref, btau_ref,
                   h_ref, y_ref, tau_ref, hsum_ref):
    i = pl.program_id(0)
    s = jnp.maximum(s0_ref[...], s1_ref[...])
    aggr = jnp.maximum(a_ref[...] + bm_ref[...] + s, 0.0)
    h = jnp.maximum(
        jnp.dot(z_ref[...], wua_ref[...], preferred_element_type=jnp.float32)
        + jnp.dot(aggr, wub_ref[...], preferred_element_type=jnp.float32)
        + bu_ref[...], 0.0)
    h_ref[...] = h
    y_ref[...] = (
        jnp.dot(z_ref[...], wdz_ref[...], preferred_element_type=jnp.float32)
        + jnp.dot(h, wdh_ref[...], preferred_element_type=jnp.float32)
        + bdec_ref[...])

    @pl.when(i == 0)
    def _():
        hsum_ref[...] = jnp.zeros_like(hsum_ref)

    hsum_ref[...] += jnp.sum(h, axis=0, keepdims=True)

    @pl.when(i == pl.num_programs(0) - 1)
    def _():
        hmean = hsum_ref[...] * (1.0 / NN)
        tau_ref[...] = (
            jnp.dot(hmean, wtau_ref[...], preferred_element_type=jnp.float32)
            + btau_ref[...])


def kernel(x, pre_h, edge_index, edge_attr, W_enc, b_enc, W_M, b_M, W_U, b_U,
           W_dec, b_dec, W_tau, b_tau):
    src = edge_index[0]
    dst = edge_index[1]

    wxa = W_enc[:, :DD].T
    wxb = W_enc[:, DD:].T
    wma = W_M[:, :DD].T
    wmb = W_M[:, DD:2 * DD].T
    wme = W_M[:, 2 * DD:].T
    wua = W_U[:, :DD].T
    wub = W_U[:, DD:].T
    wdz = W_dec[:, :DD].T
    wdh = W_dec[:, DD:].T
    wtau = W_tau.T

    n_blocks = NN // ROW_BLK
    row_spec = pl.BlockSpec((ROW_BLK, DD), lambda i: (i, 0))
    full_spec = pl.BlockSpec((DD, DD), lambda i: (0, 0))
    bias_spec = pl.BlockSpec((1, DD), lambda i: (0, 0))

    z, a_tab, b_tab = pl.pallas_call(
        _enc_ab_body,
        grid=(n_blocks,),
        in_specs=[row_spec, row_spec, full_spec, full_spec, bias_spec,
                  full_spec, full_spec],
        out_specs=[row_spec, row_spec, row_spec],
        out_shape=[jax.ShapeDtypeStruct((NN, DD), jnp.float32)] * 3,
    )(x, pre_h, wxa, wxb, b_enc.reshape(1, DD), wma, wmb)

    ea_proj = pl.pallas_call(
        _ea_body,
        grid=(EE // EA_BLK,),
        in_specs=[pl.BlockSpec((EA_BLK, EDIMK), lambda i: (i, 0)),
                  pl.BlockSpec((EDIMK, DD), lambda i: (0, 0))],
        out_specs=pl.BlockSpec((EA_BLK, DD), lambda i: (i, 0)),
        out_shape=jax.ShapeDtypeStruct((EE, DD), jnp.float32),
    )(edge_attr, wme)

    s_pad = _sc_segmax(dst, src, b_tab, ea_proj)
    s0 = s_pad[0, :NN]
    s1 = s_pad[1, :NN]

    col_spec = pl.BlockSpec((DD, 1), lambda i: (0, 0))
    h, y, tau = pl.pallas_call(
        _epilogue_body,
        grid=(n_blocks,),
        in_specs=[row_spec, row_spec, row_spec, row_spec, bias_spec,
                  full_spec, full_spec, bias_spec,
                  col_spec, col_spec, pl.BlockSpec((1, 1), lambda i: (0, 0)),
                  col_spec, pl.BlockSpec((1, 1), lambda i: (0, 0))],
        out_specs=[row_spec, pl.BlockSpec((ROW_BLK, 1), lambda i: (i, 0)),
                   pl.BlockSpec((1, 1), lambda i: (0, 0))],
        out_shape=[jax.ShapeDtypeStruct((NN, DD), jnp.float32),
                   jax.ShapeDtypeStruct((NN, 1), jnp.float32),
                   jax.ShapeDtypeStruct((1, 1), jnp.float32)],
        scratch_shapes=[pltpu_vmem((1, DD), jnp.float32)],
    )(z, a_tab, s0, s1, b_M.reshape(1, DD), wua, wub, b_U.reshape(1, DD),
      wdz, wdh, b_dec.reshape(1, 1), wtau, b_tau.reshape(1, 1))

    return (h, y, tau)


def pltpu_vmem(shape, dtype):
    from jax.experimental.pallas import tpu as pltpu
    return pltpu.VMEM(shape, dtype)
